# initial kernel scaffold (unmeasured)
import jax
import jax.numpy as jnp
from jax import lax
from jax.experimental import pallas as pl
from jax.experimental.pallas import tpu as pltpu

N_DEV = 4
N_TOK = 2048
D_MODEL = 512
H = 1024
N_EXP = 16
E_LOC = 4
CAP = 102
CAP_PAD = 128
ROWS = E_LOC * CAP_PAD
M_OUT = N_TOK // N_DEV


def _body(xg_ref, w_ref, owner_ref, local_ref, out_ref,
          ymat_ref, dest_ref, recv_ref, send_sems, recv_sems):
    my = lax.axis_index("i")

    bar = pltpu.get_barrier_semaphore()
    for k in range(1, N_DEV):
        pl.semaphore_signal(
            bar, inc=1,
            device_id=((my + k) % N_DEV,),
            device_id_type=pl.DeviceIdType.MESH,
        )
    pl.semaphore_wait(bar, N_DEV - 1)

    for e in range(E_LOC):
        ymat_ref[e * CAP_PAD:(e + 1) * CAP_PAD, :] = jnp.dot(
            xg_ref[e * CAP_PAD:(e + 1) * CAP_PAD, :],
            w_ref[e],
            preferred_element_type=jnp.float32,
        )

    dest_ref[...] = jnp.zeros_like(dest_ref)

    def scat(r, carry):
        o = owner_ref[r]
        l = local_ref[r]
        dest_ref[o, pl.ds(l, 1), :] = ymat_ref[pl.ds(r, 1), :]
        return carry

    lax.fori_loop(0, ROWS, scat, 0)

    rdmas = []
    for j in range(1, N_DEV):
        t = (my + j) % N_DEV
        rdma = pltpu.make_async_remote_copy(
            src_ref=dest_ref.at[t],
            dst_ref=recv_ref.at[3 - j],
            send_sem=send_sems.at[j - 1],
            recv_sem=recv_sems.at[3 - j],
            device_id=(t,),
            device_id_type=pl.DeviceIdType.MESH,
        )
        rdma.start()
        rdmas.append(rdma)
    for rdma in rdmas:
        rdma.wait()

    out_ref[...] = (
        dest_ref[my] + recv_ref[0] + recv_ref[1] + recv_ref[2]
    )


def kernel(x, router_W, route_idx, expert_W):
    del router_W
    my = lax.axis_index("i")

    idx = route_idx[:, 0].astype(jnp.int32)
    onehot = (idx[:, None] == jnp.arange(N_EXP, dtype=jnp.int32)[None, :])
    pos = jnp.cumsum(onehot.astype(jnp.int32), axis=0) - 1
    rank = jnp.take_along_axis(pos, idx[:, None], axis=1)[:, 0]
    admit = rank < CAP

    e_loc = idx - my * E_LOC
    mine = admit & (e_loc >= 0) & (e_loc < E_LOC)

    le_s = jnp.where(mine, e_loc, E_LOC)
    c_s = jnp.where(mine, rank, CAP_PAD)
    tok = jnp.full((E_LOC, CAP_PAD), N_TOK, jnp.int32)
    tok = tok.at[le_s, c_s].set(jnp.arange(N_TOK, dtype=jnp.int32), mode="drop")
    tok_flat = tok.reshape(ROWS)

    x_g = x[jnp.clip(tok_flat, 0, N_TOK - 1)]

    owner = tok_flat // M_OUT
    local = tok_flat % M_OUT

    return pl.pallas_call(
        _body,
        out_shape=jax.ShapeDtypeStruct((M_OUT, H), jnp.float32),
        in_specs=[
            pl.BlockSpec(memory_space=pltpu.VMEM),
            pl.BlockSpec(memory_space=pltpu.VMEM),
            pl.BlockSpec(memory_space=pltpu.SMEM),
            pl.BlockSpec(memory_space=pltpu.SMEM),
        ],
        out_specs=pl.BlockSpec(memory_space=pltpu.VMEM),
        scratch_shapes=[
            pltpu.VMEM((ROWS, H), jnp.float32),
            pltpu.VMEM((N_DEV + 1, M_OUT, H), jnp.float32),
            pltpu.VMEM((N_DEV - 1, M_OUT, H), jnp.float32),
            pltpu.SemaphoreType.DMA((N_DEV - 1,)),
            pltpu.SemaphoreType.DMA((N_DEV - 1,)),
        ],
        compiler_params=pltpu.CompilerParams(collective_id=0),
    )(x_g, expert_W, owner, local)


# baseline (device time: 70396 ns/iter reference)
import jax
import jax.numpy as jnp
from jax import lax
from jax.experimental import pallas as pl
from jax.experimental.pallas import tpu as pltpu

N_DEV = 4
N_TOK = 2048
D_MODEL = 512
H = 1024
N_EXP = 16
E_LOC = 4
CAP = 102
M_OUT = N_TOK // N_DEV


def _body(x_ref, w_ref, mask_ref, out_ref, contrib_ref, recv_ref,
          send_sems, recv_sems):
    my = lax.axis_index("i")

    bar = pltpu.get_barrier_semaphore()
    for k in range(1, N_DEV):
        pl.semaphore_signal(
            bar, inc=1,
            device_id=((my + k) % N_DEV,),
            device_id_type=pl.DeviceIdType.MESH,
        )
    pl.semaphore_wait(bar, N_DEV - 1)

    x = x_ref[...]
    contrib_ref[...] = jnp.dot(
        x * mask_ref[:, 0:1], w_ref[0], preferred_element_type=jnp.float32
    )
    for le in range(1, E_LOC):
        contrib_ref[...] += jnp.dot(
            x * mask_ref[:, le:le + 1], w_ref[le],
            preferred_element_type=jnp.float32,
        )

    rdmas = []
    for j in range(1, N_DEV):
        t = (my + j) % N_DEV
        rdma = pltpu.make_async_remote_copy(
            src_ref=contrib_ref.at[pl.ds(t * M_OUT, M_OUT)],
            dst_ref=recv_ref.at[3 - j],
            send_sem=send_sems.at[j - 1],
            recv_sem=recv_sems.at[3 - j],
            device_id=(t,),
            device_id_type=pl.DeviceIdType.MESH,
        )
        rdma.start()
        rdmas.append(rdma)
    for rdma in rdmas:
        rdma.wait()

    out_ref[...] = (
        contrib_ref[pl.ds(my * M_OUT, M_OUT), :]
        + recv_ref[0] + recv_ref[1] + recv_ref[2]
    )


def kernel(x, router_W, route_idx, expert_W):
    del router_W
    my = lax.axis_index("i")

    idx = route_idx[:, 0].astype(jnp.int32)
    oh4 = idx[:, None] == (
        my * E_LOC + jnp.arange(E_LOC, dtype=jnp.int32)
    )[None, :]
    cum4 = jnp.cumsum(oh4.astype(jnp.int32), axis=0)
    mask = (oh4 & (cum4 <= CAP)).astype(jnp.float32)

    return pl.pallas_call(
        _body,
        out_shape=jax.ShapeDtypeStruct((M_OUT, H), jnp.float32),
        in_specs=[
            pl.BlockSpec(memory_space=pltpu.VMEM),
            pl.BlockSpec(memory_space=pltpu.VMEM),
            pl.BlockSpec(memory_space=pltpu.VMEM),
        ],
        out_specs=pl.BlockSpec(memory_space=pltpu.VMEM),
        scratch_shapes=[
            pltpu.VMEM((N_TOK, H), jnp.float32),
            pltpu.VMEM((N_DEV - 1, M_OUT, H), jnp.float32),
            pltpu.SemaphoreType.DMA((N_DEV - 1,)),
            pltpu.SemaphoreType.DMA((N_DEV - 1,)),
        ],
        compiler_params=pltpu.CompilerParams(collective_id=0),
    )(x, expert_W, mask)


# device time: 66115 ns/iter; 1.0648x vs baseline; 1.0648x over previous
import jax
import jax.numpy as jnp
from jax import lax
from jax.experimental import pallas as pl
from jax.experimental.pallas import tpu as pltpu

N_DEV = 4
N_TOK = 2048
D_MODEL = 512
H = 1024
N_EXP = 16
E_LOC = 4
CAP = 102
M_OUT = N_TOK // N_DEV


def _body(x_ref, w_ref, mask_ref, out_ref, dest_ref, recv_ref,
          send_sems, recv_sems):
    my = lax.axis_index("i")

    bar = pltpu.get_barrier_semaphore()
    for k in range(1, N_DEV):
        pl.semaphore_signal(
            bar, inc=1,
            device_id=((my + k) % N_DEV,),
            device_id_type=pl.DeviceIdType.MESH,
        )
    pl.semaphore_wait(bar, N_DEV - 1)

    def block(t):
        xb = x_ref[pl.ds(t * M_OUT, M_OUT), :]
        mb = mask_ref[pl.ds(t * M_OUT, M_OUT), :]
        acc = jnp.dot(
            xb * mb[:, 0:1], w_ref[0], preferred_element_type=jnp.float32
        )
        for le in range(1, E_LOC):
            acc += jnp.dot(
                xb * mb[:, le:le + 1], w_ref[le],
                preferred_element_type=jnp.float32,
            )
        return acc

    rdmas = []
    for j in range(1, N_DEV):
        t = (my + j) % N_DEV
        dest_ref[j - 1] = block(t)
        rdma = pltpu.make_async_remote_copy(
            src_ref=dest_ref.at[j - 1],
            dst_ref=recv_ref.at[3 - j],
            send_sem=send_sems.at[j - 1],
            recv_sem=recv_sems.at[3 - j],
            device_id=(t,),
            device_id_type=pl.DeviceIdType.MESH,
        )
        rdma.start()
        rdmas.append(rdma)

    out_ref[...] = block(my)

    for rdma in rdmas:
        rdma.wait()

    out_ref[...] += recv_ref[0] + recv_ref[1] + recv_ref[2]


def kernel(x, router_W, route_idx, expert_W):
    del router_W
    my = lax.axis_index("i")

    idx = route_idx[:, 0].astype(jnp.int32)
    oh4 = idx[:, None] == (
        my * E_LOC + jnp.arange(E_LOC, dtype=jnp.int32)
    )[None, :]
    cum4 = jnp.cumsum(oh4.astype(jnp.int32), axis=0)
    mask = (oh4 & (cum4 <= CAP)).astype(jnp.float32)

    return pl.pallas_call(
        _body,
        out_shape=jax.ShapeDtypeStruct((M_OUT, H), jnp.float32),
        in_specs=[
            pl.BlockSpec(memory_space=pltpu.VMEM),
            pl.BlockSpec(memory_space=pltpu.VMEM),
            pl.BlockSpec(memory_space=pltpu.VMEM),
        ],
        out_specs=pl.BlockSpec(memory_space=pltpu.VMEM),
        scratch_shapes=[
            pltpu.VMEM((N_DEV - 1, M_OUT, H), jnp.float32),
            pltpu.VMEM((N_DEV - 1, M_OUT, H), jnp.float32),
            pltpu.SemaphoreType.DMA((N_DEV - 1,)),
            pltpu.SemaphoreType.DMA((N_DEV - 1,)),
        ],
        compiler_params=pltpu.CompilerParams(collective_id=0),
    )(x, expert_W, mask)


# device time: 43699 ns/iter; 1.6109x vs baseline; 1.5130x over previous
import jax
import jax.numpy as jnp
from jax import lax
from jax.experimental import pallas as pl
from jax.experimental.pallas import tpu as pltpu

N_DEV = 4
N_TOK = 2048
D_MODEL = 512
H = 1024
N_EXP = 16
E_LOC = 4
CAP = 102
M_OUT = N_TOK // N_DEV


def _body(x_ref, w_ref, mask_ref, out_ref, dest_ref, recv_ref,
          send_sems, recv_sems):
    my = lax.axis_index("i")

    bar = pltpu.get_barrier_semaphore()
    for k in range(1, N_DEV):
        pl.semaphore_signal(
            bar, inc=1,
            device_id=((my + k) % N_DEV,),
            device_id_type=pl.DeviceIdType.MESH,
        )
    pl.semaphore_wait(bar, N_DEV - 1)

    def block(t):
        xb = x_ref[pl.ds(t * M_OUT, M_OUT), :]
        mb = mask_ref[pl.ds(t * M_OUT, M_OUT), :]
        acc = jnp.dot(
            xb * mb[:, 0:1], w_ref[0], preferred_element_type=jnp.float32
        )
        for le in range(1, E_LOC):
            acc += jnp.dot(
                xb * mb[:, le:le + 1], w_ref[le],
                preferred_element_type=jnp.float32,
            )
        return acc

    rdmas = []
    for j in range(1, N_DEV):
        t = (my + j) % N_DEV
        dest_ref[j - 1] = block(t).astype(jnp.bfloat16)
        rdma = pltpu.make_async_remote_copy(
            src_ref=dest_ref.at[j - 1],
            dst_ref=recv_ref.at[3 - j],
            send_sem=send_sems.at[j - 1],
            recv_sem=recv_sems.at[3 - j],
            device_id=(t,),
            device_id_type=pl.DeviceIdType.MESH,
        )
        rdma.start()
        rdmas.append(rdma)

    out_ref[...] = block(my)

    for rdma in rdmas:
        rdma.wait()

    out_ref[...] += (
        recv_ref[0].astype(jnp.float32)
        + recv_ref[1].astype(jnp.float32)
        + recv_ref[2].astype(jnp.float32)
    )


def kernel(x, router_W, route_idx, expert_W):
    del router_W
    my = lax.axis_index("i")

    idx = route_idx[:, 0].astype(jnp.int32)
    oh4 = idx[:, None] == (
        my * E_LOC + jnp.arange(E_LOC, dtype=jnp.int32)
    )[None, :]
    cum4 = jnp.cumsum(oh4.astype(jnp.int32), axis=0)
    mask = (oh4 & (cum4 <= CAP)).astype(jnp.float32)

    return pl.pallas_call(
        _body,
        out_shape=jax.ShapeDtypeStruct((M_OUT, H), jnp.float32),
        in_specs=[
            pl.BlockSpec(memory_space=pltpu.VMEM),
            pl.BlockSpec(memory_space=pltpu.VMEM),
            pl.BlockSpec(memory_space=pltpu.VMEM),
        ],
        out_specs=pl.BlockSpec(memory_space=pltpu.VMEM),
        scratch_shapes=[
            pltpu.VMEM((N_DEV - 1, M_OUT, H), jnp.bfloat16),
            pltpu.VMEM((N_DEV - 1, M_OUT, H), jnp.bfloat16),
            pltpu.SemaphoreType.DMA((N_DEV - 1,)),
            pltpu.SemaphoreType.DMA((N_DEV - 1,)),
        ],
        compiler_params=pltpu.CompilerParams(collective_id=0),
    )(x, expert_W, mask)
